# baseline (speedup 1.0000x reference)
import jax
import jax.numpy as jnp
from jax import lax
from jax.experimental import pallas as pl
from jax.experimental.pallas import tpu as pltpu
from jax.experimental.pallas import tpu_sc as plsc

VOCAB = 100000
NUM_FRAME = 16384
TVS_DIM = 64
LANES = 16

NC, NS = 2, 16
NW = NC * NS
FPW = TVS_DIM // NW
CH = 2048
NCHUNK = NUM_FRAME // CH


def _sc_body(z_hbm, means_t, stds_t, noise_hbm, out_hbm,
             row_v, colm_v, zc_v, nzc_v, sem_row, sem_in, sem_out):
    wid = lax.axis_index("s") * NC + lax.axis_index("c")

    def stage_z(ch):
        return pltpu.async_copy(
            z_hbm.at[pl.ds(ch * CH, CH)], zc_v.at[ch % 2], sem_in)

    def stage_n(ch):
        return pltpu.async_copy(
            noise_hbm.at[pl.ds(ch * CH, CH)], nzc_v.at[ch % 2], sem_in)

    pending = []
    for k in range(FPW):
        f = wid * FPW + k

        hrow = pltpu.async_copy(means_t.at[f], row_v, sem_row)
        hz = stage_z(0)
        for h in pending:
            h.wait()
        pending = []
        hrow.wait()

        for ch in range(NCHUNK):
            hz.wait()
            if ch + 1 < NCHUNK:
                hz = stage_z(ch + 1)

            @plsc.parallel_loop(0, CH // LANES, unroll=8)
            def ga(g, ch=ch):
                z16 = zc_v[ch % 2, pl.ds(g * LANES, LANES)]
                colm_v[pl.ds(ch * CH + g * LANES, LANES)] = (
                    plsc.load_gather(row_v, [z16]))

        hrow = pltpu.async_copy(stds_t.at[f], row_v, sem_row)
        hz = stage_z(0)
        hn = stage_n(0)
        hrow.wait()

        for ch in range(NCHUNK):
            hz.wait()
            hn.wait()
            if ch + 1 < NCHUNK:
                hz = stage_z(ch + 1)
                hn = stage_n(ch + 1)

            @plsc.parallel_loop(0, CH // LANES, unroll=8)
            def gb(g, ch=ch):
                sl = pl.ds(g * LANES, LANES)
                pos = pl.ds(ch * CH + g * LANES, LANES)
                z16 = zc_v[ch % 2, sl]
                s16 = plsc.load_gather(row_v, [z16])
                m16 = colm_v[pos]
                n16 = nzc_v[ch % 2, sl]
                colm_v[pos] = jnp.clip(m16 + n16 * s16, -1.0, 1.0)

            pending.append(pltpu.async_copy(
                colm_v.at[pl.ds(ch * CH, CH)],
                out_hbm.at[f, pl.ds(ch * CH, CH)], sem_out))

    for h in pending:
        h.wait()


@jax.jit
def kernel(z, target_means, target_stds, noise):
    z1 = z.astype(jnp.int32)
    noise1 = noise.reshape(NUM_FRAME)
    means_t = target_means.T
    stds_t = target_stds.T

    mesh = plsc.VectorSubcoreMesh(
        core_axis_name="c", subcore_axis_name="s",
        num_cores=NC, num_subcores=NS)
    run = pl.kernel(
        _sc_body,
        mesh=mesh,
        out_type=jax.ShapeDtypeStruct((TVS_DIM, NUM_FRAME), jnp.float32),
        scratch_types=[
            pltpu.VMEM((VOCAB,), jnp.float32),
            pltpu.VMEM((NUM_FRAME,), jnp.float32),
            pltpu.VMEM((2, CH), jnp.int32),
            pltpu.VMEM((2, CH), jnp.float32),
            pltpu.SemaphoreType.DMA,
            pltpu.SemaphoreType.DMA,
            pltpu.SemaphoreType.DMA,
        ],
        compiler_params=pltpu.CompilerParams(needs_layout_passes=False),
    )
    return run(z1, means_t, stds_t, noise1).T


# speedup vs baseline: 1.0227x; 1.0227x over previous
import jax
import jax.numpy as jnp
from jax import lax
from jax.experimental import pallas as pl
from jax.experimental.pallas import tpu as pltpu
from jax.experimental.pallas import tpu_sc as plsc

VOCAB = 100000
NUM_FRAME = 16384
TVS_DIM = 64
LANES = 16

NC, NS = 2, 16
NW = NC * NS
FPW = TVS_DIM // NW
CH = 4096
NCHUNK = NUM_FRAME // CH


def _sc_body(z_hbm, means_t, stds_t, noise_hbm, out_hbm,
             row_v, colm_v, zc_v, nzc_v, sem):
    wid = lax.axis_index("s") * NC + lax.axis_index("c")

    pending = []
    for k in range(FPW):
        f = wid * FPW + k

        pltpu.sync_copy(means_t.at[f], row_v)
        for h in pending:
            h.wait()
        pending = []

        for ch in range(NCHUNK):
            pltpu.sync_copy(z_hbm.at[pl.ds(ch * CH, CH)], zc_v)

            @plsc.parallel_loop(0, CH // LANES, unroll=16)
            def ga(g, ch=ch):
                z16 = zc_v[pl.ds(g * LANES, LANES)]
                colm_v[pl.ds(ch * CH + g * LANES, LANES)] = (
                    plsc.load_gather(row_v, [z16]))

        pltpu.sync_copy(stds_t.at[f], row_v)

        for ch in range(NCHUNK):
            pltpu.sync_copy(z_hbm.at[pl.ds(ch * CH, CH)], zc_v)
            pltpu.sync_copy(noise_hbm.at[pl.ds(ch * CH, CH)], nzc_v)

            @plsc.parallel_loop(0, CH // LANES, unroll=16)
            def gb(g, ch=ch):
                sl = pl.ds(g * LANES, LANES)
                pos = pl.ds(ch * CH + g * LANES, LANES)
                z16 = zc_v[sl]
                s16 = plsc.load_gather(row_v, [z16])
                m16 = colm_v[pos]
                n16 = nzc_v[sl]
                colm_v[pos] = jnp.clip(m16 + n16 * s16, -1.0, 1.0)
            pending.append(pltpu.async_copy(
                colm_v.at[pl.ds(ch * CH, CH)],
                out_hbm.at[f, pl.ds(ch * CH, CH)], sem))

    for h in pending:
        h.wait()


@jax.jit
def kernel(z, target_means, target_stds, noise):
    z1 = z.astype(jnp.int32)
    noise1 = noise.reshape(NUM_FRAME)
    means_t = target_means.T
    stds_t = target_stds.T

    mesh = plsc.VectorSubcoreMesh(
        core_axis_name="c", subcore_axis_name="s",
        num_cores=NC, num_subcores=NS)
    run = pl.kernel(
        _sc_body,
        mesh=mesh,
        out_type=jax.ShapeDtypeStruct((TVS_DIM, NUM_FRAME), jnp.float32),
        scratch_types=[
            pltpu.VMEM((VOCAB,), jnp.float32),
            pltpu.VMEM((NUM_FRAME,), jnp.float32),
            pltpu.VMEM((CH,), jnp.int32),
            pltpu.VMEM((CH,), jnp.float32),
            pltpu.SemaphoreType.DMA,
        ],
        compiler_params=pltpu.CompilerParams(needs_layout_passes=False),
    )
    return run(z1, means_t, stds_t, noise1).T


# speedup vs baseline: 1.0430x; 1.0199x over previous
import jax
import jax.numpy as jnp
from jax import lax
from jax.experimental import pallas as pl
from jax.experimental.pallas import tpu as pltpu
from jax.experimental.pallas import tpu_sc as plsc

VOCAB = 100000
NUM_FRAME = 16384
TVS_DIM = 64
LANES = 16

NC, NS = 2, 16
NW = NC * NS
FPW = TVS_DIM // NW
CH = 4096
NCHUNK = NUM_FRAME // CH


def _sc_body(z_hbm, means_t, stds_t, noise_hbm, out_hbm,
             row_v, colm_v, zc_v, nzc_v, sem):
    wid = lax.axis_index("s") * NC + lax.axis_index("c")

    pending = []
    for k in range(FPW):
        f = wid * FPW + k

        pltpu.sync_copy(means_t.at[f], row_v)
        for h in pending:
            h.wait()
        pending = []

        for ch in range(NCHUNK):
            pltpu.sync_copy(z_hbm.at[pl.ds(ch * CH, CH)], zc_v)

            @plsc.parallel_loop(0, CH // LANES, unroll=8)
            def ga(g, ch=ch):
                z16 = zc_v[pl.ds(g * LANES, LANES)]
                colm_v[pl.ds(ch * CH + g * LANES, LANES)] = (
                    plsc.load_gather(row_v, [z16]))

        pltpu.sync_copy(stds_t.at[f], row_v)

        for ch in range(NCHUNK):
            pltpu.sync_copy(z_hbm.at[pl.ds(ch * CH, CH)], zc_v)
            pltpu.sync_copy(noise_hbm.at[pl.ds(ch * CH, CH)], nzc_v)

            @plsc.parallel_loop(0, CH // LANES, unroll=8)
            def gb(g, ch=ch):
                sl = pl.ds(g * LANES, LANES)
                pos = pl.ds(ch * CH + g * LANES, LANES)
                z16 = zc_v[sl]
                s16 = plsc.load_gather(row_v, [z16])
                m16 = colm_v[pos]
                n16 = nzc_v[sl]
                colm_v[pos] = jnp.clip(m16 + n16 * s16, -1.0, 1.0)
            pending.append(pltpu.async_copy(
                colm_v.at[pl.ds(ch * CH, CH)],
                out_hbm.at[f, pl.ds(ch * CH, CH)], sem))

    for h in pending:
        h.wait()


@jax.jit
def kernel(z, target_means, target_stds, noise):
    z1 = z.astype(jnp.int32)
    noise1 = noise.reshape(NUM_FRAME)
    means_t = target_means.T
    stds_t = target_stds.T

    mesh = plsc.VectorSubcoreMesh(
        core_axis_name="c", subcore_axis_name="s",
        num_cores=NC, num_subcores=NS)
    run = pl.kernel(
        _sc_body,
        mesh=mesh,
        out_type=jax.ShapeDtypeStruct((TVS_DIM, NUM_FRAME), jnp.float32),
        scratch_types=[
            pltpu.VMEM((VOCAB,), jnp.float32),
            pltpu.VMEM((NUM_FRAME,), jnp.float32),
            pltpu.VMEM((CH,), jnp.int32),
            pltpu.VMEM((CH,), jnp.float32),
            pltpu.SemaphoreType.DMA,
        ],
        compiler_params=pltpu.CompilerParams(needs_layout_passes=False),
    )
    return run(z1, means_t, stds_t, noise1).T


# speedup vs baseline: 1.1392x; 1.0923x over previous
import jax
import jax.numpy as jnp
from jax import lax
from jax.experimental import pallas as pl
from jax.experimental.pallas import tpu as pltpu
from jax.experimental.pallas import tpu_sc as plsc

VOCAB = 100000
NUM_FRAME = 16384
TVS_DIM = 64
LANES = 16

NC, NS = 2, 16
NW = NC * NS
FPW = TVS_DIM // NW
CH = 4096
NCHUNK = NUM_FRAME // CH


def _sc_body(z_hbm, means_t, stds_t, noise_hbm, out_hbm,
             row_v, colm_v, zc_v, nzc_v, sem):
    wid = lax.axis_index("s") * NC + lax.axis_index("c")

    pending = []
    for k in range(FPW):
        f = wid * FPW + k

        pltpu.sync_copy(means_t.at[f], row_v)
        for h in pending:
            h.wait()
        pending = []

        for ch in range(NCHUNK):
            if ch % 2 == 0:
                pltpu.sync_copy(z_hbm.at[pl.ds(ch * CH, 2 * CH)], zc_v)

            @plsc.parallel_loop(0, CH // LANES, unroll=8)
            def ga(g, ch=ch):
                z16 = zc_v[pl.ds((ch % 2) * CH + g * LANES, LANES)]
                colm_v[pl.ds(ch * CH + g * LANES, LANES)] = (
                    plsc.load_gather(row_v, [z16]))

        pltpu.sync_copy(stds_t.at[f], row_v)

        for ch in range(NCHUNK):
            if ch % 2 == 0:
                pltpu.sync_copy(z_hbm.at[pl.ds(ch * CH, 2 * CH)], zc_v)
            pltpu.sync_copy(noise_hbm.at[pl.ds(ch * CH, CH)], nzc_v)

            @plsc.parallel_loop(0, CH // LANES, unroll=8)
            def gb(g, ch=ch):
                sl = pl.ds(g * LANES, LANES)
                pos = pl.ds(ch * CH + g * LANES, LANES)
                z16 = zc_v[pl.ds((ch % 2) * CH + g * LANES, LANES)]
                s16 = plsc.load_gather(row_v, [z16])
                m16 = colm_v[pos]
                n16 = nzc_v[sl]
                colm_v[pos] = jnp.clip(m16 + n16 * s16, -1.0, 1.0)
        pending.append(pltpu.async_copy(colm_v, out_hbm.at[f], sem))

    for h in pending:
        h.wait()


@jax.jit
def kernel(z, target_means, target_stds, noise):
    z1 = z.astype(jnp.int32)
    noise1 = noise.reshape(NUM_FRAME)
    means_t = target_means.T
    stds_t = target_stds.T

    mesh = plsc.VectorSubcoreMesh(
        core_axis_name="c", subcore_axis_name="s",
        num_cores=NC, num_subcores=NS)
    run = pl.kernel(
        _sc_body,
        mesh=mesh,
        out_type=jax.ShapeDtypeStruct((TVS_DIM, NUM_FRAME), jnp.float32),
        scratch_types=[
            pltpu.VMEM((VOCAB,), jnp.float32),
            pltpu.VMEM((NUM_FRAME,), jnp.float32),
            pltpu.VMEM((2 * CH,), jnp.int32),
            pltpu.VMEM((CH,), jnp.float32),
            pltpu.SemaphoreType.DMA,
        ],
        compiler_params=pltpu.CompilerParams(needs_layout_passes=False),
    )
    return run(z1, means_t, stds_t, noise1).T
